# Initial kernel scaffold; baseline (speedup 1.0000x reference)
import jax
import jax.numpy as jnp
from jax.experimental import pallas as pl


def kernel(x, edge_index, batch, W1, b1, Ws1, bs1, Ws2, bs2, W2, b2, Wf, bf):
    raise NotImplementedError("write your pallas kernel here")


# speedup vs baseline: 27.6065x; 27.6065x over previous
import functools

import jax
import jax.numpy as jnp
from jax import lax
from jax.experimental import pallas as pl
from jax.experimental.pallas import tpu as pltpu
from jax.experimental.pallas import tpu_sc as plsc

_N = 50000
_E = 1600000
_G = 512
_NP = 50176
_LAN = 32
_EP = 1605632
_EROWS = _EP // 128
_RPT = _EROWS // 32
_NPAIR = _RPT // 2
_SNR = _NP // 16
_TNR = _NP // 32
_GP = 520

_f32 = jnp.float32
_i32 = jnp.int32

@functools.lru_cache(maxsize=None)
def _mesh():
    return plsc.VectorSubcoreMesh(
        core_axis_name="c", subcore_axis_name="s", num_cores=2, num_subcores=16)


def _sc_pass_body(v, srcs, dsts, out, spacc, isrc, idst, rows, zbuf, gs0, gs1):
    c = lax.axis_index("c")
    s = lax.axis_index("s")
    wid = s * 2 + c

    def _zrow(r, carry):
        zbuf[r, 0:16] = jnp.zeros((16,), _f32)
        zbuf[r, 16:32] = jnp.zeros((16,), _f32)
        return carry
    lax.fori_loop(0, 224, _zrow, 0)

    def _zcp(i, carry):
        pltpu.sync_copy(zbuf, spacc.at[pl.ds(s * _SNR + i * 224, 224)])
        return carry
    lax.fori_loop(0, 14, _zcp, 0)
    plsc.subcore_barrier()

    ebase = wid * _RPT

    def _pair(p, carry):
        k0 = 2 * p
        blk = k0 // 8
        slot = lax.rem(blk, 2)
        @pl.when(lax.rem(p, 4) == 0)
        def _load_idx():
            pltpu.sync_copy(srcs.at[pl.ds(ebase + blk * 8, 8)],
                            isrc.at[pl.ds(slot * 8, 8)])
            pltpu.sync_copy(dsts.at[pl.ds(ebase + blk * 8, 8)],
                            idst.at[pl.ds(slot * 8, 8)])
        r0 = slot * 8 + lax.rem(k0, 8)
        r1 = slot * 8 + lax.rem(k0 + 1, 8)
        pltpu.async_copy(v.at[isrc.at[r0]], rows.at[0], gs0)
        @pl.when(p > 0)
        def _finish_prev():
            pk = k0 - 1
            pr = lax.rem(pk // 8, 2) * 8 + lax.rem(pk, 8)
            pltpu.make_async_copy(v.at[isrc.at[pr]], rows.at[1], gs1).wait()
            pltpu.sync_copy(rows.at[1], spacc.at[idst.at[pr]], add=True)
        pltpu.async_copy(v.at[isrc.at[r1]], rows.at[1], gs1)
        pltpu.make_async_copy(v.at[isrc.at[r0]], rows.at[0], gs0).wait()
        pltpu.sync_copy(rows.at[0], spacc.at[idst.at[r0]], add=True)
        return carry
    lax.fori_loop(0, _NPAIR, _pair, 0)

    pltpu.make_async_copy(v.at[isrc.at[7]], rows.at[1], gs1).wait()
    pltpu.sync_copy(rows.at[1], spacc.at[idst.at[7]], add=True)

    plsc.subcore_barrier()
    pltpu.sync_copy(spacc.at[pl.ds(s * _SNR, _SNR)],
                    out.at[pl.ds(c * _NP + s * _SNR, _SNR)])


@functools.lru_cache(maxsize=None)
def _sc_pass_kernel():
    return pl.kernel(
        _sc_pass_body,
        out_type=jax.ShapeDtypeStruct((2 * _NP, _LAN), _f32),
        mesh=_mesh(),
        scratch_types=[
            pltpu.VMEM_SHARED((_NP, _LAN), _f32),
            pltpu.VMEM((16, 128), _i32),
            pltpu.VMEM((16, 128), _i32),
            pltpu.VMEM((2, 128, _LAN), _f32),
            pltpu.VMEM((224, _LAN), _f32),
            pltpu.SemaphoreType.DMA,
            pltpu.SemaphoreType.DMA,
        ],
        compiler_params=pltpu.CompilerParams(use_tc_tiling_on_sc=False),
    )


def _sc_pass(v, srcs, dsts):
    return _sc_pass_kernel()(v, srcs, dsts)


def _sc_pool_body(h, bat, out, pb, rbuf, ibuf):
    c = lax.axis_index("c")
    s = lax.axis_index("s")
    wid = s * 2 + c
    base = wid * _TNR

    ninf = jnp.full((16,), -jnp.inf, _f32)
    def _irow(r, carry):
        pb[r, 0:16] = ninf
        pb[r, 16:32] = ninf
        return carry
    lax.fori_loop(0, _GP, _irow, 0)

    iota16 = lax.broadcasted_iota(_i32, (16,), 0)

    def _chunk(cc, carry):
        row0 = base + cc * 112
        pltpu.sync_copy(h.at[pl.ds(row0, 112)], rbuf)
        pltpu.sync_copy(bat.at[pl.ds(row0, 112)], ibuf)
        def _grp(gi, carry2):
            bv = ibuf[pl.ds(gi * 16, 16)]
            for l in range(16):
                seg = jnp.sum(jnp.where(iota16 == l, bv, 0), axis=0)
                r = gi * 16 + l
                pb[seg, 0:16] = jnp.maximum(pb[seg, 0:16], rbuf[r, 0:16])
                pb[seg, 16:32] = jnp.maximum(pb[seg, 16:32], rbuf[r, 16:32])
            return carry2
        lax.fori_loop(0, 7, _grp, 0)
        return carry
    lax.fori_loop(0, 14, _chunk, 0)

    pltpu.sync_copy(pb, out.at[wid])


@functools.lru_cache(maxsize=None)
def _sc_pool_kernel():
    return pl.kernel(
        _sc_pool_body,
        out_type=jax.ShapeDtypeStruct((32, _GP, _LAN), _f32),
        mesh=_mesh(),
        scratch_types=[
            pltpu.VMEM((_GP, _LAN), _f32),
            pltpu.VMEM((112, _LAN), _f32),
            pltpu.VMEM((112,), _i32),
        ],
        compiler_params=pltpu.CompilerParams(
            use_tc_tiling_on_sc=False, needs_layout_passes=False),
    )


def _sc_pool(h, batchp):
    return _sc_pool_kernel()(h, batchp)


_BN = 512
_NBLK = _NP // _BN


def _row_spec():
    return pl.BlockSpec((_BN, _LAN), lambda i: (i, 0))


def _acc_spec():
    return pl.BlockSpec((2, _BN, _LAN), lambda i: (0, i, 0))


def _const_spec(shape):
    return pl.BlockSpec(shape, lambda i: tuple(0 for _ in shape))


def _prep_body(xb, w, o):
    o[...] = jnp.dot(xb[...], w[...], preferred_element_type=_f32)


_tc_prep = pl.pallas_call(
    _prep_body,
    grid=(_NBLK,),
    in_specs=[pl.BlockSpec((_BN, 8), lambda i: (i, 0)), _const_spec((8, _LAN))],
    out_specs=_row_spec(),
    out_shape=jax.ShapeDtypeStruct((_NP, _LAN), _f32),
)


def _merge_a_body(vb, ab, bb, g_o, d_o):
    u = vb[...] + ab[0] + ab[1]
    lane = lax.broadcasted_iota(_i32, (_BN, _LAN), 1)
    deg = jnp.sum(jnp.where(lane == 30, u, 0.0), axis=1, keepdims=True)
    dinv = jnp.where(deg > 0, lax.rsqrt(deg), 0.0)
    h1 = jnp.where(lane < 30, jnp.maximum(u + bb[...], 0.0), 0.0)
    g_o[...] = dinv * h1
    d_o[...] = jnp.broadcast_to(dinv, (_BN, _LAN))


_tc_merge_a = pl.pallas_call(
    _merge_a_body,
    grid=(_NBLK,),
    in_specs=[_row_spec(), _acc_spec(), _const_spec((1, _LAN))],
    out_specs=[_row_spec(), _row_spec()],
    out_shape=[jax.ShapeDtypeStruct((_NP, _LAN), _f32),
               jax.ShapeDtypeStruct((_NP, _LAN), _f32)],
)


def _merge_b_body(vb, ab, db, g_o):
    d = db[...]
    g_o[...] = (vb[...] + ab[0] + ab[1]) * (d * d)


_tc_merge_b = pl.pallas_call(
    _merge_b_body,
    grid=(_NBLK,),
    in_specs=[_row_spec(), _acc_spec(), _row_spec()],
    out_specs=_row_spec(),
    out_shape=jax.ShapeDtypeStruct((_NP, _LAN), _f32),
)


def _merge_c_body(vb, ab, db, w, bb, g_o):
    d = db[...]
    t = (vb[...] + ab[0] + ab[1]) * d
    g_o[...] = (jnp.dot(t, w[...], preferred_element_type=_f32) + bb[...]) * d


_tc_merge_c = pl.pallas_call(
    _merge_c_body,
    grid=(_NBLK,),
    in_specs=[_row_spec(), _acc_spec(), _row_spec(),
              _const_spec((_LAN, _LAN)), _const_spec((1, _LAN))],
    out_specs=_row_spec(),
    out_shape=jax.ShapeDtypeStruct((_NP, _LAN), _f32),
)


def _merge_c2_body(vb, ab, db, w1, bb1, w2, z_o):
    t = (vb[...] + ab[0] + ab[1]) * db[...]
    h3 = jnp.dot(t, w1[...], preferred_element_type=_f32) + bb1[...]
    z_o[...] = jnp.dot(h3, w2[...], preferred_element_type=_f32)


_tc_merge_c2 = pl.pallas_call(
    _merge_c2_body,
    grid=(_NBLK,),
    in_specs=[_row_spec(), _acc_spec(), _row_spec(),
              _const_spec((_LAN, _LAN)), _const_spec((1, _LAN)),
              _const_spec((_LAN, _LAN))],
    out_specs=_row_spec(),
    out_shape=jax.ShapeDtypeStruct((_NP, _LAN), _f32),
)


def _merge_d_body(vb, ab, bb, h_o):
    lane = lax.broadcasted_iota(_i32, (_BN, _LAN), 1)
    u = vb[...] + ab[0] + ab[1] + bb[...]
    h_o[...] = jnp.where(lane < 30, jnp.maximum(u, 0.0), 0.0)


_tc_merge_d = pl.pallas_call(
    _merge_d_body,
    grid=(_NBLK,),
    in_specs=[_row_spec(), _acc_spec(), _const_spec((1, _LAN))],
    out_specs=_row_spec(),
    out_shape=jax.ShapeDtypeStruct((_NP, _LAN), _f32),
)


def _head_body(pb, wf, bf_, o):
    pooled = jnp.max(pb[...], axis=0)
    p = pooled[:_G, :]
    logits = jnp.dot(p, wf[...], preferred_element_type=_f32) + bf_[...]
    lane = lax.broadcasted_iota(_i32, (_G, 128), 1)
    lm = jnp.where(lane < 3, logits, -jnp.inf)
    m = jnp.max(lm, axis=1, keepdims=True)
    e = jnp.where(lane < 3, jnp.exp(lm - m), 0.0)
    lse = jnp.log(jnp.sum(e, axis=1, keepdims=True))
    o[...] = lm - m - lse


_tc_head = pl.pallas_call(
    _head_body,
    grid=(1,),
    in_specs=[pl.BlockSpec((32, _GP, _LAN), lambda i: (0, 0, 0)),
              _const_spec((_LAN, 128)), _const_spec((1, 128))],
    out_specs=pl.BlockSpec((_G, 128), lambda i: (0, 0)),
    out_shape=jax.ShapeDtypeStruct((_G, 128), _f32),
)


def kernel(x, edge_index, batch, W1, b1, Ws1, bs1, Ws2, bs2, W2, b2, Wf, bf):
    src = edge_index[0].astype(_i32)
    dst = edge_index[1].astype(_i32)
    fill = jnp.full((_EP - _E,), _NP - 1, _i32)
    srcs = jnp.concatenate([src, fill]).reshape(_EROWS, 128)
    dsts = jnp.concatenate([dst, fill]).reshape(_EROWS, 128)
    batchp = jnp.concatenate(
        [batch.astype(_i32), jnp.full((_NP - _N,), _G, _i32)])

    xp = jnp.zeros((_NP, 8), _f32).at[:_N, :5].set(x).at[:_N, 5].set(1.0)
    W1p = jnp.zeros((8, _LAN), _f32).at[:5, :30].set(W1).at[5, 30].set(1.0)
    b1p = jnp.zeros((1, _LAN), _f32).at[0, :30].set(b1)
    Ws1p = jnp.zeros((_LAN, _LAN), _f32).at[:30, :30].set(Ws1)
    bs1p = jnp.zeros((1, _LAN), _f32).at[0, :30].set(bs1)
    Ws2p = jnp.zeros((_LAN, _LAN), _f32).at[:30, :30].set(Ws2)
    bs2p = jnp.zeros((1, _LAN), _f32).at[0, :30].set(bs2)
    W2p = jnp.zeros((_LAN, _LAN), _f32).at[:30, :30].set(W2)
    b2p = jnp.zeros((1, _LAN), _f32).at[0, :30].set(b2)
    Wfp = jnp.zeros((_LAN, 128), _f32).at[:30, :3].set(Wf)
    bfp = jnp.zeros((1, 128), _f32).at[0, :3].set(bf)

    y0 = _tc_prep(xp, W1p)

    acc = _sc_pass(y0, srcs, dsts).reshape(2, _NP, _LAN)
    g, dinvb = _tc_merge_a(y0, acc, b1p)

    for i in range(5):
        acc = _sc_pass(g, srcs, dsts).reshape(2, _NP, _LAN)
        if i < 4:
            g = _tc_merge_b(g, acc, dinvb)
    g = _tc_merge_c(g, acc, dinvb, Ws1p, bs1p)

    for i in range(5):
        acc = _sc_pass(g, srcs, dsts).reshape(2, _NP, _LAN)
        if i < 4:
            g = _tc_merge_b(g, acc, dinvb)
    z = _tc_merge_c2(g, acc, dinvb, Ws2p, bs2p, W2p)

    acc = _sc_pass(z, srcs, dsts).reshape(2, _NP, _LAN)
    h4 = _tc_merge_d(z, acc, b2p)

    parts = _sc_pool(h4, batchp)
    outp = _tc_head(parts, Wfp, bfp)
    return outp[:, :3]


# speedup vs baseline: 30.5844x; 1.1079x over previous
import functools

import jax
import jax.numpy as jnp
from jax import lax
from jax.experimental import pallas as pl
from jax.experimental.pallas import tpu as pltpu
from jax.experimental.pallas import tpu_sc as plsc

_N = 50000
_E = 1600000
_G = 512
_NP = 50176
_LAN = 32
_EP = 1605632
_EROWS = _EP // 128
_RPT = _EROWS // 32
_NPAIR = _RPT // 2
_SNR = _NP // 16
_TNR = _NP // 32
_GP = 520

_f32 = jnp.float32
_i32 = jnp.int32

@functools.lru_cache(maxsize=None)
def _mesh():
    return plsc.VectorSubcoreMesh(
        core_axis_name="c", subcore_axis_name="s", num_cores=2, num_subcores=16)


def _sc_pass_body(v, srcs, dsts, out, spacc, isrc, idst, rows, zbuf,
                  g0, g1, g2, g3, s0, s1, s2, s3):
    gs = [g0, g1, g2, g3]
    ss = [s0, s1, s2, s3]
    c = lax.axis_index("c")
    s = lax.axis_index("s")
    wid = s * 2 + c

    def _zrow(r, carry):
        zbuf[r, 0:16] = jnp.zeros((16,), _f32)
        zbuf[r, 16:32] = jnp.zeros((16,), _f32)
        return carry
    lax.fori_loop(0, 224, _zrow, 0)

    def _zcp(i, carry):
        pltpu.sync_copy(zbuf, spacc.at[pl.ds(s * _SNR + i * 224, 224)])
        return carry
    lax.fori_loop(0, 14, _zcp, 0)
    plsc.subcore_barrier()

    ebase = wid * _RPT

    def _quad(q, carry):
        k0 = 4 * q
        blk = k0 // 8
        slot = lax.rem(blk, 2)
        @pl.when(lax.rem(q, 2) == 0)
        def _load_idx():
            pltpu.sync_copy(srcs.at[pl.ds(ebase + blk * 8, 8)],
                            isrc.at[pl.ds(slot * 8, 8)])
            pltpu.sync_copy(dsts.at[pl.ds(ebase + blk * 8, 8)],
                            idst.at[pl.ds(slot * 8, 8)])
        rbase = slot * 8 + lax.rem(k0, 8)
        for j in range(4):
            @pl.when(q > 0)
            def _drain_scatter(j=j):
                pltpu.make_async_copy(
                    rows.at[j], spacc.at[idst.at[rbase + j]], ss[j]).wait()
            pltpu.async_copy(v.at[isrc.at[rbase + j]], rows.at[j], gs[j])
        for j in range(4):
            pltpu.make_async_copy(
                v.at[isrc.at[rbase + j]], rows.at[j], gs[j]).wait()
            pltpu.async_copy(
                rows.at[j], spacc.at[idst.at[rbase + j]], ss[j], add=True)
        return carry
    lax.fori_loop(0, _RPT // 4, _quad, 0)

    for j in range(4):
        pltpu.make_async_copy(rows.at[j], spacc.at[idst.at[4 + j]], ss[j]).wait()

    plsc.subcore_barrier()
    pltpu.sync_copy(spacc.at[pl.ds(s * _SNR, _SNR)],
                    out.at[pl.ds(c * _NP + s * _SNR, _SNR)])


@functools.lru_cache(maxsize=None)
def _sc_pass_kernel():
    return pl.kernel(
        _sc_pass_body,
        out_type=jax.ShapeDtypeStruct((2 * _NP, _LAN), _f32),
        mesh=_mesh(),
        scratch_types=[
            pltpu.VMEM_SHARED((_NP, _LAN), _f32),
            pltpu.VMEM((16, 128), _i32),
            pltpu.VMEM((16, 128), _i32),
            pltpu.VMEM((4, 128, _LAN), _f32),
            pltpu.VMEM((224, _LAN), _f32),
            pltpu.SemaphoreType.DMA,
            pltpu.SemaphoreType.DMA,
            pltpu.SemaphoreType.DMA,
            pltpu.SemaphoreType.DMA,
            pltpu.SemaphoreType.DMA,
            pltpu.SemaphoreType.DMA,
            pltpu.SemaphoreType.DMA,
            pltpu.SemaphoreType.DMA,
        ],
        compiler_params=pltpu.CompilerParams(use_tc_tiling_on_sc=False),
    )


def _sc_pass(v, srcs, dsts):
    return _sc_pass_kernel()(v, srcs, dsts)


def _sc_pool_body(h, bat, out, pb, rbuf, ibuf):
    c = lax.axis_index("c")
    s = lax.axis_index("s")
    wid = s * 2 + c
    base = wid * _TNR

    ninf = jnp.full((16,), -jnp.inf, _f32)
    def _irow(r, carry):
        pb[r, 0:16] = ninf
        pb[r, 16:32] = ninf
        return carry
    lax.fori_loop(0, _GP, _irow, 0)

    iota16 = lax.broadcasted_iota(_i32, (16,), 0)

    def _chunk(cc, carry):
        row0 = base + cc * 112
        pltpu.sync_copy(h.at[pl.ds(row0, 112)], rbuf)
        pltpu.sync_copy(bat.at[pl.ds(row0, 112)], ibuf)
        def _grp(gi, carry2):
            bv = ibuf[pl.ds(gi * 16, 16)]
            for l in range(16):
                seg = jnp.sum(jnp.where(iota16 == l, bv, 0), axis=0)
                r = gi * 16 + l
                pb[seg, 0:16] = jnp.maximum(pb[seg, 0:16], rbuf[r, 0:16])
                pb[seg, 16:32] = jnp.maximum(pb[seg, 16:32], rbuf[r, 16:32])
            return carry2
        lax.fori_loop(0, 7, _grp, 0)
        return carry
    lax.fori_loop(0, 14, _chunk, 0)

    pltpu.sync_copy(pb, out.at[wid])


@functools.lru_cache(maxsize=None)
def _sc_pool_kernel():
    return pl.kernel(
        _sc_pool_body,
        out_type=jax.ShapeDtypeStruct((32, _GP, _LAN), _f32),
        mesh=_mesh(),
        scratch_types=[
            pltpu.VMEM((_GP, _LAN), _f32),
            pltpu.VMEM((112, _LAN), _f32),
            pltpu.VMEM((112,), _i32),
        ],
        compiler_params=pltpu.CompilerParams(
            use_tc_tiling_on_sc=False, needs_layout_passes=False),
    )


def _sc_pool(h, batchp):
    return _sc_pool_kernel()(h, batchp)


_BN = 512
_NBLK = _NP // _BN


def _row_spec():
    return pl.BlockSpec((_BN, _LAN), lambda i: (i, 0))


def _acc_spec():
    return pl.BlockSpec((2, _BN, _LAN), lambda i: (0, i, 0))


def _const_spec(shape):
    return pl.BlockSpec(shape, lambda i: tuple(0 for _ in shape))


def _prep_body(xb, w, o):
    o[...] = jnp.dot(xb[...], w[...], preferred_element_type=_f32)


_tc_prep = pl.pallas_call(
    _prep_body,
    grid=(_NBLK,),
    in_specs=[pl.BlockSpec((_BN, 8), lambda i: (i, 0)), _const_spec((8, _LAN))],
    out_specs=_row_spec(),
    out_shape=jax.ShapeDtypeStruct((_NP, _LAN), _f32),
)


def _merge_a_body(vb, ab, bb, g_o, d_o):
    u = vb[...] + ab[0] + ab[1]
    lane = lax.broadcasted_iota(_i32, (_BN, _LAN), 1)
    deg = jnp.sum(jnp.where(lane == 30, u, 0.0), axis=1, keepdims=True)
    dinv = jnp.where(deg > 0, lax.rsqrt(deg), 0.0)
    h1 = jnp.where(lane < 30, jnp.maximum(u + bb[...], 0.0), 0.0)
    g_o[...] = dinv * h1
    d_o[...] = jnp.broadcast_to(dinv, (_BN, _LAN))


_tc_merge_a = pl.pallas_call(
    _merge_a_body,
    grid=(_NBLK,),
    in_specs=[_row_spec(), _acc_spec(), _const_spec((1, _LAN))],
    out_specs=[_row_spec(), _row_spec()],
    out_shape=[jax.ShapeDtypeStruct((_NP, _LAN), _f32),
               jax.ShapeDtypeStruct((_NP, _LAN), _f32)],
)


def _merge_b_body(vb, ab, db, g_o):
    d = db[...]
    g_o[...] = (vb[...] + ab[0] + ab[1]) * (d * d)


_tc_merge_b = pl.pallas_call(
    _merge_b_body,
    grid=(_NBLK,),
    in_specs=[_row_spec(), _acc_spec(), _row_spec()],
    out_specs=_row_spec(),
    out_shape=jax.ShapeDtypeStruct((_NP, _LAN), _f32),
)


def _merge_c_body(vb, ab, db, w, bb, g_o):
    d = db[...]
    t = (vb[...] + ab[0] + ab[1]) * d
    g_o[...] = (jnp.dot(t, w[...], preferred_element_type=_f32) + bb[...]) * d


_tc_merge_c = pl.pallas_call(
    _merge_c_body,
    grid=(_NBLK,),
    in_specs=[_row_spec(), _acc_spec(), _row_spec(),
              _const_spec((_LAN, _LAN)), _const_spec((1, _LAN))],
    out_specs=_row_spec(),
    out_shape=jax.ShapeDtypeStruct((_NP, _LAN), _f32),
)


def _merge_c2_body(vb, ab, db, w1, bb1, w2, z_o):
    t = (vb[...] + ab[0] + ab[1]) * db[...]
    h3 = jnp.dot(t, w1[...], preferred_element_type=_f32) + bb1[...]
    z_o[...] = jnp.dot(h3, w2[...], preferred_element_type=_f32)


_tc_merge_c2 = pl.pallas_call(
    _merge_c2_body,
    grid=(_NBLK,),
    in_specs=[_row_spec(), _acc_spec(), _row_spec(),
              _const_spec((_LAN, _LAN)), _const_spec((1, _LAN)),
              _const_spec((_LAN, _LAN))],
    out_specs=_row_spec(),
    out_shape=jax.ShapeDtypeStruct((_NP, _LAN), _f32),
)


def _merge_d_body(vb, ab, bb, h_o):
    lane = lax.broadcasted_iota(_i32, (_BN, _LAN), 1)
    u = vb[...] + ab[0] + ab[1] + bb[...]
    h_o[...] = jnp.where(lane < 30, jnp.maximum(u, 0.0), 0.0)


_tc_merge_d = pl.pallas_call(
    _merge_d_body,
    grid=(_NBLK,),
    in_specs=[_row_spec(), _acc_spec(), _const_spec((1, _LAN))],
    out_specs=_row_spec(),
    out_shape=jax.ShapeDtypeStruct((_NP, _LAN), _f32),
)


def _head_body(pb, wf, bf_, o):
    pooled = jnp.max(pb[...], axis=0)
    p = pooled[:_G, :]
    logits = jnp.dot(p, wf[...], preferred_element_type=_f32) + bf_[...]
    lane = lax.broadcasted_iota(_i32, (_G, 128), 1)
    lm = jnp.where(lane < 3, logits, -jnp.inf)
    m = jnp.max(lm, axis=1, keepdims=True)
    e = jnp.where(lane < 3, jnp.exp(lm - m), 0.0)
    lse = jnp.log(jnp.sum(e, axis=1, keepdims=True))
    o[...] = lm - m - lse


_tc_head = pl.pallas_call(
    _head_body,
    grid=(1,),
    in_specs=[pl.BlockSpec((32, _GP, _LAN), lambda i: (0, 0, 0)),
              _const_spec((_LAN, 128)), _const_spec((1, 128))],
    out_specs=pl.BlockSpec((_G, 128), lambda i: (0, 0)),
    out_shape=jax.ShapeDtypeStruct((_G, 128), _f32),
)


def kernel(x, edge_index, batch, W1, b1, Ws1, bs1, Ws2, bs2, W2, b2, Wf, bf):
    src = edge_index[0].astype(_i32)
    dst = edge_index[1].astype(_i32)
    fill = jnp.full((_EP - _E,), _NP - 1, _i32)
    srcs = jnp.concatenate([src, fill]).reshape(_EROWS, 128)
    dsts = jnp.concatenate([dst, fill]).reshape(_EROWS, 128)
    batchp = jnp.concatenate(
        [batch.astype(_i32), jnp.full((_NP - _N,), _G, _i32)])

    xp = jnp.zeros((_NP, 8), _f32).at[:_N, :5].set(x).at[:_N, 5].set(1.0)
    W1p = jnp.zeros((8, _LAN), _f32).at[:5, :30].set(W1).at[5, 30].set(1.0)
    b1p = jnp.zeros((1, _LAN), _f32).at[0, :30].set(b1)
    Ws1p = jnp.zeros((_LAN, _LAN), _f32).at[:30, :30].set(Ws1)
    bs1p = jnp.zeros((1, _LAN), _f32).at[0, :30].set(bs1)
    Ws2p = jnp.zeros((_LAN, _LAN), _f32).at[:30, :30].set(Ws2)
    bs2p = jnp.zeros((1, _LAN), _f32).at[0, :30].set(bs2)
    W2p = jnp.zeros((_LAN, _LAN), _f32).at[:30, :30].set(W2)
    b2p = jnp.zeros((1, _LAN), _f32).at[0, :30].set(b2)
    Wfp = jnp.zeros((_LAN, 128), _f32).at[:30, :3].set(Wf)
    bfp = jnp.zeros((1, 128), _f32).at[0, :3].set(bf)

    y0 = _tc_prep(xp, W1p)

    acc = _sc_pass(y0, srcs, dsts).reshape(2, _NP, _LAN)
    g, dinvb = _tc_merge_a(y0, acc, b1p)

    for i in range(5):
        acc = _sc_pass(g, srcs, dsts).reshape(2, _NP, _LAN)
        if i < 4:
            g = _tc_merge_b(g, acc, dinvb)
    g = _tc_merge_c(g, acc, dinvb, Ws1p, bs1p)

    for i in range(5):
        acc = _sc_pass(g, srcs, dsts).reshape(2, _NP, _LAN)
        if i < 4:
            g = _tc_merge_b(g, acc, dinvb)
    z = _tc_merge_c2(g, acc, dinvb, Ws2p, bs2p, W2p)

    acc = _sc_pass(z, srcs, dsts).reshape(2, _NP, _LAN)
    h4 = _tc_merge_d(z, acc, b2p)

    parts = _sc_pool(h4, batchp)
    outp = _tc_head(parts, Wfp, bfp)
    return outp[:, :3]
